# baseline (speedup 1.0000x reference)
import dataclasses
import functools
import math

import jax
import jax.numpy as jnp
from jax import lax
from jax.experimental import pallas as pl
from jax.experimental.pallas import tpu as pltpu
from jax.experimental.pallas import tpu_sc as plsc

N = 10000
E = 320000
D = 128
AVG_LOG = math.log(33.0)

NSUB = 64
NODES_PER_S = 160
NPAD = NSUB * NODES_PER_S
CHUNK = 256
EPAD = E + 8 * CHUNK
LANES = 16
NF = D // LANES
BIG = 3.0e38


def _extract(vec, mask, zero):
    return jax.lax.reduce_sum_p.bind(
        jnp.where(mask, vec, zero), axes=(0,))


def _sc_segment_kernel(b_hbm, ssrc_hbm, sdst_hbm, wb_hbm, out_hbm,
                       idx0_v, idx1_v, rows0_v, rows1_v, dst0_v, dst1_v,
                       slab_v, wb_v, sem0, sem1):
    w = lax.axis_index("s") * 2 + lax.axis_index("c")
    pltpu.sync_copy(wb_hbm, wb_v)
    idx_b = (idx0_v, idx1_v)
    rows_b = (rows0_v, rows1_v)
    dst_b = (dst0_v, dst1_v)
    sem_b = (sem0, sem1)

    zeros = jnp.zeros((LANES,), jnp.float32)
    bigs = jnp.full((LANES,), BIG, jnp.float32)
    izero = jnp.zeros((LANES,), jnp.int32)
    lane = lax.iota(jnp.int32, LANES)
    masks = [lane == k for k in range(LANES)]

    for sub in range(2):
        g = w * 2 + sub
        n0 = g * NODES_PER_S
        gq = pl.multiple_of((g // LANES) * LANES, LANES)
        gr = g % LANES
        wvec = wb_v[pl.ds(gq, LANES)]
        wvec2 = wb_v[pl.ds(gq + LANES, LANES)]
        gmask = lane == gr
        gmask2 = lane == (gr + 1)
        e_lo = _extract(wvec, gmask, izero)
        e_hi = jnp.where(
            gr == LANES - 1,
            _extract(wvec2, masks[0], izero),
            _extract(wvec, gmask2, izero))
        base0 = (e_lo // CHUNK) * CHUNK
        nchunks = (e_hi - base0 + (CHUNK - 1)) // CHUNK
        npairs = jnp.maximum((nchunks + 1) // 2, 1)

        def flush(cur, accs):
            r = cur - n0
            owned = (r >= 0) & (r < NODES_PER_S)
            row = jnp.where(owned, r, NODES_PER_S)
            off = pl.multiple_of(row * (3 * D), LANES)
            for k in range(3 * NF):
                slab_v[pl.ds(off + k * LANES, LANES)] = accs[k]

        def issue(c, buf):
            base = base0 + c * CHUNK
            pltpu.sync_copy(ssrc_hbm.at[pl.ds(base, CHUNK)], idx_b[buf])
            pltpu.sync_copy(sdst_hbm.at[pl.ds(base, CHUNK)], dst_b[buf])
            pltpu.async_copy(b_hbm.at[idx_b[buf]], rows_b[buf], sem_b[buf])

        def wait(buf):
            pltpu.make_async_copy(b_hbm.at[idx_b[buf]], rows_b[buf],
                                  sem_b[buf]).wait()

        def compute(c, buf, carry):
            rows_v = rows_b[buf]
            dst_v = dst_b[buf]

            def group_body(gi, gcarry):
                dvec = dst_v[pl.ds(gi * LANES, LANES)]
                cur = gcarry[0]
                accs = list(gcarry[1:])
                for k in range(LANES):
                    j = gi * LANES + k
                    d = _extract(dvec, masks[k], izero)
                    new_seg = d != cur

                    @pl.when(new_seg)
                    def _():
                        flush(cur, accs)

                    rrow = rows_v.at[j]
                    for f in range(NF):
                        if f % 2 == 0:
                            vi = rrow[pl.ds((f // 2) * LANES, LANES)]
                            v = lax.bitcast_convert_type(
                                lax.shift_left(vi, 16), jnp.float32)
                        else:
                            v = lax.bitcast_convert_type(
                                vi & jnp.int32(-65536), jnp.float32)
                        accs[f] = jnp.where(new_seg, v, accs[f] + v)
                        accs[NF + f] = jnp.where(
                            new_seg, v, jnp.minimum(accs[NF + f], v))
                        accs[2 * NF + f] = jnp.where(
                            new_seg, v, jnp.maximum(accs[2 * NF + f], v))
                    cur = d
                return (cur,) + tuple(accs)

            return lax.fori_loop(0, CHUNK // LANES, group_body, carry)

        issue(0, 0)

        def pair_body(cp, carry):
            c0 = 2 * cp
            issue(c0 + 1, 1)
            wait(0)
            carry = compute(c0, 0, carry)
            issue(c0 + 2, 0)
            wait(1)
            return compute(c0 + 1, 1, carry)

        init = (n0 + NODES_PER_S,) + tuple(zeros for _ in range(3 * NF))
        final = lax.fori_loop(0, npairs, pair_body, init)
        wait(0)
        flush(final[0], final[1:])

        pltpu.sync_copy(slab_v.at[pl.ds(0, NODES_PER_S * 3 * D)],
                        out_hbm.at[pl.ds(n0 * 3 * D, NODES_PER_S * 3 * D)])


def _sc_segment(b, ssrc_pad, sdst_pad, wb_pad):
    mesh = plsc.VectorSubcoreMesh(core_axis_name="c", subcore_axis_name="s")
    cp = pltpu.CompilerParams()
    if "needs_layout_passes" in pltpu.CompilerParams.__dataclass_fields__:
        cp = dataclasses.replace(cp, needs_layout_passes=False)
    cp = dataclasses.replace(cp, use_tc_tiling_on_sc=False)
    kern = functools.partial(
        pl.kernel,
        compiler_params=cp,
        out_type=jax.ShapeDtypeStruct((NPAD * 3 * D,), jnp.float32),
        mesh=mesh,
        scratch_types=[
            pltpu.VMEM((CHUNK,), jnp.int32),
            pltpu.VMEM((CHUNK,), jnp.int32),
            pltpu.VMEM((CHUNK, D // 2), jnp.int32),
            pltpu.VMEM((CHUNK, D // 2), jnp.int32),
            pltpu.VMEM((CHUNK,), jnp.int32),
            pltpu.VMEM((CHUNK,), jnp.int32),
            pltpu.VMEM(((NODES_PER_S + 1) * 3 * D,), jnp.float32),
            pltpu.VMEM((80,), jnp.int32),
            pltpu.SemaphoreType.DMA,
            pltpu.SemaphoreType.DMA,
        ],
    )(_sc_segment_kernel)
    return kern(b, ssrc_pad, sdst_pad, wb_pad).reshape(NPAD, 3 * D)[:N]


ROWS_BLK = 1000
_HIGH = lax.Precision.HIGHEST


def _pre_tc_kernel(h_ref, wd_ref, ws_ref, b_ref, a_ref, bout_ref):
    h = h_ref[...]
    a_ref[...] = jax.lax.dot_general(h, wd_ref[...], (((1,), (0,)), ((), ())),
                                     precision=_HIGH) + b_ref[...]
    bout_ref[...] = jax.lax.dot_general(
        h, ws_ref[...], (((1,), (0,)), ((), ())),
        precision=_HIGH).astype(jnp.bfloat16)


def _pre_tc(h, wd, ws, pre_b):
    grid = (N // ROWS_BLK,)
    return pl.pallas_call(
        _pre_tc_kernel,
        grid=grid,
        in_specs=[
            pl.BlockSpec((ROWS_BLK, D), lambda i: (i, 0)),
            pl.BlockSpec((D, D), lambda i: (0, 0)),
            pl.BlockSpec((D, D), lambda i: (0, 0)),
            pl.BlockSpec((1, D), lambda i: (0, 0)),
        ],
        out_specs=[
            pl.BlockSpec((ROWS_BLK, D), lambda i: (i, 0)),
            pl.BlockSpec((ROWS_BLK, D), lambda i: (i, 0)),
        ],
        out_shape=[
            jax.ShapeDtypeStruct((N, D), jnp.float32),
            jax.ShapeDtypeStruct((N, D), jnp.bfloat16),
        ],
    )(h, wd, ws, pre_b.reshape(1, D))


def _post_tc_kernel(h_ref, a_ref, seg_ref, r0_ref, r1_ref, pw_ref, pb_ref,
                    lw_ref, lb_ref, out_ref, *, relu):
    h = h_ref[...]
    a = a_ref[...]
    seg = seg_ref[...]
    deg = (r1_ref[...] - r0_ref[...]).astype(jnp.float32)
    has = deg > 0.0
    s = jnp.where(has, deg * a + seg[:, 0:D], 0.0)
    mn = jnp.where(has, a + seg[:, D:2 * D], 0.0)
    mx = jnp.where(has, a + seg[:, 2 * D:3 * D], 0.0)
    deg_c = jnp.maximum(deg, 1.0)
    mean = s / deg_c
    agg = jnp.concatenate([s, mean, mn, mx], axis=-1)
    log_deg = jnp.log(deg_c + 1.0)
    amp = agg * (log_deg / AVG_LOG)
    att = agg * (AVG_LOG / log_deg)
    cat = jnp.concatenate([h, agg, amp, att], axis=-1)
    t = jax.lax.dot_general(cat, pw_ref[...], (((1,), (0,)), ((), ())),
                            precision=_HIGH) + pb_ref[...]
    o = jax.lax.dot_general(t, lw_ref[...], (((1,), (0,)), ((), ())),
                            precision=_HIGH) + lb_ref[...]
    if relu:
        o = jnp.maximum(o, 0.0)
    out_ref[...] = o


def _post_tc(h, a, seg, r0, r1, post_w, post_b, lin_w, lin_b, relu):
    grid = (N // ROWS_BLK,)
    kern = functools.partial(_post_tc_kernel, relu=relu)
    return pl.pallas_call(
        kern,
        grid=grid,
        in_specs=[
            pl.BlockSpec((ROWS_BLK, D), lambda i: (i, 0)),
            pl.BlockSpec((ROWS_BLK, D), lambda i: (i, 0)),
            pl.BlockSpec((ROWS_BLK, 3 * D), lambda i: (i, 0)),
            pl.BlockSpec((ROWS_BLK, 1), lambda i: (i, 0)),
            pl.BlockSpec((ROWS_BLK, 1), lambda i: (i, 0)),
            pl.BlockSpec((13 * D, D), lambda i: (0, 0)),
            pl.BlockSpec((1, D), lambda i: (0, 0)),
            pl.BlockSpec((D, D), lambda i: (0, 0)),
            pl.BlockSpec((1, D), lambda i: (0, 0)),
        ],
        out_specs=pl.BlockSpec((ROWS_BLK, D), lambda i: (i, 0)),
        out_shape=jax.ShapeDtypeStruct((N, D), jnp.float32),
    )(h, a, seg, r0, r1, post_w, post_b.reshape(1, D), lin_w,
      lin_b.reshape(1, D))


def kernel(x, edge_index, params):
    src = edge_index[0]
    dst = edge_index[1]
    perm = jnp.argsort(dst)
    sdst = jnp.take(dst, perm)
    ssrc = jnp.take(src, perm)
    rowptr = jnp.searchsorted(sdst, jnp.arange(N + 1, dtype=jnp.int32),
                              side="left").astype(jnp.int32)
    bnds = jnp.minimum(
        jnp.arange(NSUB + 1, dtype=jnp.int32) * NODES_PER_S, N)
    wb = jnp.take(rowptr, bnds)
    wb_pad = jnp.concatenate([wb, jnp.zeros((80 - NSUB - 1,), jnp.int32)])
    ssrc_pad = jnp.concatenate([ssrc, jnp.zeros((EPAD - E,), jnp.int32)])
    sdst_pad = jnp.concatenate(
        [sdst, jnp.full((EPAD - E,), NPAD, jnp.int32)])
    r0 = rowptr[:-1].reshape(N, 1)
    r1 = rowptr[1:].reshape(N, 1)

    perm = jnp.array(
        [32 * (m // 2) + 2 * t + (m % 2)
         for m in range(NF) for t in range(LANES)], dtype=jnp.int32)
    row_idx = jnp.concatenate(
        [jnp.arange(D, dtype=jnp.int32)]
        + [b * D + perm for b in range(1, 13)])

    h = x
    for l, p in enumerate(params):
        wd = jnp.take(p["pre_W"][:D], perm, axis=1)
        ws = p["pre_W"][D:]
        pre_b = jnp.take(p["pre_b"], perm)
        post_w = jnp.take(p["post_W"], row_idx, axis=0)
        a, b = _pre_tc(h, wd, ws, pre_b)
        b_i32 = lax.bitcast_convert_type(
            b.reshape(N, D // 2, 2), jnp.int32)
        seg = _sc_segment(b_i32, ssrc_pad, sdst_pad, wb_pad)
        h = _post_tc(h, a, seg, r0, r1, post_w, p["post_b"],
                     p["lin_W"], p["lin_b"], relu=(l < len(params) - 1))
    return h
